# baseline (speedup 1.0000x reference)
import functools

import jax
import jax.numpy as jnp
from jax import lax
from jax.experimental import pallas as pl
from jax.experimental.pallas import tpu as pltpu
from jax.experimental.pallas import tpu_sc as plsc

N = 10000
E = 320000
NC, NS = 2, 16
NW = NC * NS
PT = E // NW
W = 8
CH = 10
CL = PT // CH
DEG_C = 2000
RP = N // NS

f32 = jnp.float32


def _mesh():
    return plsc.VectorSubcoreMesh(
        core_axis_name="c", subcore_axis_name="s", num_cores=NC, num_subcores=NS
    )


_SC_PARAMS = pltpu.CompilerParams(use_tc_tiling_on_sc=False)


@functools.partial(
    pl.kernel,
    mesh=_mesh(),
    out_type=jax.ShapeDtypeStruct((NC, N, W), f32),
    compiler_params=_SC_PARAMS,
    scratch_types=[
        pltpu.VMEM((PT // DEG_C, DEG_C), jnp.int32),
        pltpu.VMEM((DEG_C, W), f32),
        pltpu.VMEM_SHARED((N, W), f32),
        pltpu.SemaphoreType.DMA,
    ],
)
def _deg_kernel(ei_hbm, zeros_hbm, ones_hbm, out_hbm, idx_v, ones_v, acc_sh, sem):
    cid = lax.axis_index("c")
    sid = lax.axis_index("s")
    wid = sid * NC + cid
    rows = pl.ds(sid * RP, RP)

    pltpu.sync_copy(zeros_hbm.at[rows], acc_sh.at[rows])
    pltpu.sync_copy(ones_hbm, ones_v)
    for k in range(PT // DEG_C):
        pltpu.sync_copy(
            ei_hbm.at[1].at[pl.ds(wid * PT + k * DEG_C, DEG_C)], idx_v.at[k])
    plsc.subcore_barrier()

    for k in range(PT // DEG_C):
        pltpu.async_copy(ones_v, acc_sh.at[idx_v.at[k]], sem, add=True)
    for k in range(PT // DEG_C):
        pltpu.make_async_copy(ones_v, acc_sh.at[idx_v.at[k]], sem).wait()

    plsc.subcore_barrier()
    pltpu.sync_copy(acc_sh.at[rows], out_hbm.at[cid].at[rows])


@functools.partial(
    pl.kernel,
    mesh=_mesh(),
    out_type=jax.ShapeDtypeStruct((NC, N, W), f32),
    compiler_params=_SC_PARAMS,
    scratch_types=[
        pltpu.VMEM((CH, CL), jnp.int32),
        pltpu.VMEM((CH, CL), jnp.int32),
        pltpu.VMEM((2, CL, W), f32),
        pltpu.VMEM_SHARED((N, W), f32),
        pltpu.VMEM_SHARED((N, W), f32),
        pltpu.SemaphoreType.DMA,
        pltpu.SemaphoreType.DMA,
        pltpu.SemaphoreType.DMA,
        pltpu.SemaphoreType.DMA,
    ],
)
def _scat_kernel(ei_hbm, g_hbm, zeros_hbm, out_hbm,
                 src_v, dst_v, msg_v, g_sh, acc_sh, gs0, gs1, ss0, ss1):
    cid = lax.axis_index("c")
    sid = lax.axis_index("s")
    wid = sid * NC + cid
    rows = pl.ds(sid * RP, RP)
    gsem = (gs0, gs1)
    ssem = (ss0, ss1)

    pltpu.sync_copy(zeros_hbm.at[rows], acc_sh.at[rows])
    pltpu.sync_copy(g_hbm.at[rows], g_sh.at[rows])
    for c in range(CH):
        pltpu.sync_copy(
            ei_hbm.at[0].at[pl.ds(wid * PT + c * CL, CL)], src_v.at[c])
        pltpu.sync_copy(
            ei_hbm.at[1].at[pl.ds(wid * PT + c * CL, CL)], dst_v.at[c])
    plsc.subcore_barrier()

    def gather(c, b):
        pltpu.async_copy(g_sh.at[src_v.at[c]], msg_v.at[b], gsem[b])

    def scatter(c, b):
        pltpu.async_copy(msg_v.at[b], acc_sh.at[dst_v.at[c]], ssem[b], add=True)

    gather(0, 0)
    for c in range(CH):
        b = c % 2
        pltpu.make_async_copy(g_sh.at[src_v.at[c]], msg_v.at[b], gsem[b]).wait()
        scatter(c, b)
        if c + 1 < CH:
            nb = 1 - b
            if c >= 1:
                pltpu.make_async_copy(
                    msg_v.at[nb], acc_sh.at[dst_v.at[c - 1]], ssem[nb]).wait()
            gather(c + 1, nb)
    pltpu.make_async_copy(
        msg_v.at[(CH - 1) % 2], acc_sh.at[dst_v.at[CH - 1]],
        ssem[(CH - 1) % 2]).wait()
    pltpu.make_async_copy(
        msg_v.at[(CH - 2) % 2], acc_sh.at[dst_v.at[CH - 2]],
        ssem[(CH - 2) % 2]).wait()

    plsc.subcore_barrier()
    pltpu.sync_copy(acc_sh.at[rows], out_hbm.at[cid].at[rows])


def _tc1(x, W1, deg):
    def body(x_ref, w_ref, deg_ref, dis_ref, g_ref):
        h = jnp.dot(x_ref[...], w_ref[...], preferred_element_type=f32)
        dis = lax.rsqrt(deg_ref[0][:, :1] + deg_ref[1][:, :1] + 1.0)
        dis_ref[...] = dis
        g_ref[...] = jnp.concatenate([h * dis, jnp.zeros((N, W - 4), f32)], axis=1)

    return pl.pallas_call(
        body,
        out_shape=(
            jax.ShapeDtypeStruct((N, 1), f32),
            jax.ShapeDtypeStruct((N, W), f32),
        ),
    )(x, W1, deg)


def _tc2(dis, g1, S1, b1, W2):
    def body(dis_ref, g_ref, s_ref, b_ref, w_ref, g2_ref):
        acc = (g_ref[:, :4] + s_ref[0][:, :4] + s_ref[1][:, :4])
        act = jnp.maximum(dis_ref[...] * acc + b_ref[...], 0.0)
        h2 = jnp.dot(act, w_ref[...], preferred_element_type=f32)
        g2_ref[...] = jnp.concatenate(
            [h2 * dis_ref[...], jnp.zeros((N, W - 2), f32)], axis=1)

    return pl.pallas_call(
        body,
        out_shape=jax.ShapeDtypeStruct((N, W), f32),
    )(dis, g1, S1, b1, W2)


def _tc3(dis, g2, S2, b2):
    def body(dis_ref, g_ref, s_ref, b_ref, out_ref):
        acc = g_ref[:, :2] + s_ref[0][:, :2] + s_ref[1][:, :2]
        out_ref[...] = dis_ref[...] * acc + b_ref[...]

    return pl.pallas_call(
        body,
        out_shape=jax.ShapeDtypeStruct((N, 2), f32),
    )(dis, g2, S2, b2)


def kernel(x, edge_index, W1, b1, W2, b2):
    ei = edge_index.astype(jnp.int32)

    zeros_w = jnp.zeros((N, W), f32)
    ones_c = jnp.ones((DEG_C, W), f32)
    deg = _deg_kernel(ei, zeros_w, ones_c)

    dis, g1 = _tc1(x, W1, deg)
    S1 = _scat_kernel(ei, g1, zeros_w)
    g2 = _tc2(dis, g1, S1, b1.reshape(1, 4), W2)
    S2 = _scat_kernel(ei, g2, zeros_w)
    out = _tc3(dis, g2, S2, b2.reshape(1, 2))
    return out


# speedup vs baseline: 1.1609x; 1.1609x over previous
import functools

import jax
import jax.numpy as jnp
from jax import lax
from jax.experimental import pallas as pl
from jax.experimental.pallas import tpu as pltpu
from jax.experimental.pallas import tpu_sc as plsc

N = 10000
E = 320000
NC, NS = 2, 16
NW = NC * NS
PT = E // NW
W = 8
CH = 10
CL = PT // CH
DEG_C = 2000
RP = N // NS

f32 = jnp.float32


def _mesh():
    return plsc.VectorSubcoreMesh(
        core_axis_name="c", subcore_axis_name="s", num_cores=NC, num_subcores=NS
    )


_SC_PARAMS = pltpu.CompilerParams(use_tc_tiling_on_sc=False)


@functools.partial(
    pl.kernel,
    mesh=_mesh(),
    out_type=jax.ShapeDtypeStruct((NC, N, W), f32),
    compiler_params=_SC_PARAMS,
    scratch_types=[
        pltpu.VMEM((PT // DEG_C, DEG_C), jnp.int32),
        pltpu.VMEM((DEG_C, W), f32),
        pltpu.VMEM_SHARED((N, W), f32),
        pltpu.SemaphoreType.DMA,
    ],
)
def _deg_kernel(ei_hbm, zeros_hbm, ones_hbm, out_hbm, idx_v, ones_v, acc_sh, sem):
    cid = lax.axis_index("c")
    sid = lax.axis_index("s")
    wid = sid * NC + cid
    rows = pl.ds(sid * RP, RP)

    pltpu.sync_copy(zeros_hbm.at[rows], acc_sh.at[rows])
    pltpu.sync_copy(ones_hbm, ones_v)
    pltpu.sync_copy(ei_hbm.at[1].at[wid], idx_v)
    plsc.subcore_barrier()

    for k in range(PT // DEG_C):
        pltpu.async_copy(ones_v, acc_sh.at[idx_v.at[k]], sem, add=True)
    for k in range(PT // DEG_C):
        pltpu.make_async_copy(ones_v, acc_sh.at[idx_v.at[k]], sem).wait()

    plsc.subcore_barrier()
    pltpu.sync_copy(acc_sh.at[rows], out_hbm.at[cid].at[rows])


@functools.partial(
    pl.kernel,
    mesh=_mesh(),
    out_type=jax.ShapeDtypeStruct((NC, N, W), f32),
    compiler_params=_SC_PARAMS,
    scratch_types=[
        pltpu.VMEM((CH, CL), jnp.int32),
        pltpu.VMEM((CH, CL), jnp.int32),
        pltpu.VMEM((2, CL, W), f32),
        pltpu.VMEM_SHARED((N, W), f32),
        pltpu.VMEM_SHARED((N, W), f32),
        pltpu.SemaphoreType.DMA,
        pltpu.SemaphoreType.DMA,
        pltpu.SemaphoreType.DMA,
        pltpu.SemaphoreType.DMA,
    ],
)
def _scat_kernel(ei_hbm, g_hbm, zeros_hbm, out_hbm,
                 src_v, dst_v, msg_v, g_sh, acc_sh, gs0, gs1, ss0, ss1):
    cid = lax.axis_index("c")
    sid = lax.axis_index("s")
    wid = sid * NC + cid
    rows = pl.ds(sid * RP, RP)
    gsem = (gs0, gs1)
    ssem = (ss0, ss1)

    pltpu.sync_copy(zeros_hbm.at[rows], acc_sh.at[rows])
    pltpu.sync_copy(g_hbm.at[rows], g_sh.at[rows])
    pltpu.sync_copy(ei_hbm.at[0].at[wid], src_v)
    pltpu.sync_copy(ei_hbm.at[1].at[wid], dst_v)
    plsc.subcore_barrier()

    def gather(c, b):
        pltpu.async_copy(g_sh.at[src_v.at[c]], msg_v.at[b], gsem[b])

    def scatter(c, b):
        pltpu.async_copy(msg_v.at[b], acc_sh.at[dst_v.at[c]], ssem[b], add=True)

    gather(0, 0)
    for c in range(CH):
        b = c % 2
        pltpu.make_async_copy(g_sh.at[src_v.at[c]], msg_v.at[b], gsem[b]).wait()
        scatter(c, b)
        if c + 1 < CH:
            nb = 1 - b
            if c >= 1:
                pltpu.make_async_copy(
                    msg_v.at[nb], acc_sh.at[dst_v.at[c - 1]], ssem[nb]).wait()
            gather(c + 1, nb)
    pltpu.make_async_copy(
        msg_v.at[(CH - 1) % 2], acc_sh.at[dst_v.at[CH - 1]],
        ssem[(CH - 1) % 2]).wait()
    pltpu.make_async_copy(
        msg_v.at[(CH - 2) % 2], acc_sh.at[dst_v.at[CH - 2]],
        ssem[(CH - 2) % 2]).wait()

    plsc.subcore_barrier()
    pltpu.sync_copy(acc_sh.at[rows], out_hbm.at[cid].at[rows])


def _tc1(x, W1, deg):
    def body(x_ref, w_ref, deg_ref, dis_ref, g_ref):
        h = jnp.dot(x_ref[...], w_ref[...], preferred_element_type=f32)
        dis = lax.rsqrt(deg_ref[0][:, :1] + deg_ref[1][:, :1] + 1.0)
        dis_ref[...] = dis
        g_ref[...] = jnp.concatenate([h * dis, jnp.zeros((N, W - 4), f32)], axis=1)

    return pl.pallas_call(
        body,
        out_shape=(
            jax.ShapeDtypeStruct((N, 1), f32),
            jax.ShapeDtypeStruct((N, W), f32),
        ),
    )(x, W1, deg)


def _tc2(dis, g1, S1, b1, W2):
    def body(dis_ref, g_ref, s_ref, b_ref, w_ref, g2_ref):
        acc = (g_ref[:, :4] + s_ref[0][:, :4] + s_ref[1][:, :4])
        act = jnp.maximum(dis_ref[...] * acc + b_ref[...], 0.0)
        h2 = jnp.dot(act, w_ref[...], preferred_element_type=f32)
        g2_ref[...] = jnp.concatenate(
            [h2 * dis_ref[...], jnp.zeros((N, W - 2), f32)], axis=1)

    return pl.pallas_call(
        body,
        out_shape=jax.ShapeDtypeStruct((N, W), f32),
    )(dis, g1, S1, b1, W2)


def _tc3(dis, g2, S2, b2):
    def body(dis_ref, g_ref, s_ref, b_ref, out_ref):
        acc = g_ref[:, :2] + s_ref[0][:, :2] + s_ref[1][:, :2]
        out_ref[...] = dis_ref[...] * acc + b_ref[...]

    return pl.pallas_call(
        body,
        out_shape=jax.ShapeDtypeStruct((N, 2), f32),
    )(dis, g2, S2, b2)


def kernel(x, edge_index, W1, b1, W2, b2):
    ei = edge_index.astype(jnp.int32)
    ei4 = ei.reshape(2, NW, CH, CL)
    eid = ei.reshape(2, NW, PT // DEG_C, DEG_C)

    zeros_w = jnp.zeros((N, W), f32)
    ones_c = jnp.ones((DEG_C, W), f32)
    deg = _deg_kernel(eid, zeros_w, ones_c)

    dis, g1 = _tc1(x, W1, deg)
    S1 = _scat_kernel(ei4, g1, zeros_w)
    g2 = _tc2(dis, g1, S1, b1.reshape(1, 4), W2)
    S2 = _scat_kernel(ei4, g2, zeros_w)
    out = _tc3(dis, g2, S2, b2.reshape(1, 2))
    return out
